# baseline (speedup 1.0000x reference)
import functools

import jax
import jax.numpy as jnp
from jax.experimental import pallas as pl
from jax.experimental.pallas import tpu as pltpu


def _gem_planes_kernel(x_ref, o_ref, acc_ref, *, khw, eps, inv_hw, inv_p):
    k = pl.program_id(1)
    last = pl.num_programs(1) - 1

    def body(i, acc):
        x = jnp.maximum(x_ref[i], jnp.float32(eps))
        return acc + x * x * x

    partial_sum = jax.lax.fori_loop(
        0, khw, body, jnp.zeros(o_ref.shape, jnp.float32), unroll=True)

    @pl.when(k == 0)
    def _init():
        acc_ref[...] = partial_sum

    @pl.when(k != 0)
    def _accum():
        acc_ref[...] += partial_sum

    @pl.when(k == last)
    def _finalize():
        o_ref[...] = jnp.power(acc_ref[...] * jnp.float32(inv_hw),
                               jnp.float32(inv_p))


def _gem(x, p=3.0, eps=1e-6):
    N, C, H, W = x.shape
    HW = H * W
    xt = jnp.transpose(x, (2, 3, 0, 1)).reshape(HW, N, C)

    bn = max(N // 2, 1)
    khw = W
    grid = (N // bn, HW // khw)

    kernel_fn = functools.partial(
        _gem_planes_kernel, khw=khw, eps=float(eps), inv_hw=1.0 / float(HW),
        inv_p=1.0 / float(p))
    out = pl.pallas_call(
        kernel_fn,
        out_shape=jax.ShapeDtypeStruct((N, C), x.dtype),
        grid=grid,
        in_specs=[pl.BlockSpec((khw, bn, C), lambda i, k: (k, i, 0))],
        out_specs=pl.BlockSpec((bn, C), lambda i, k: (i, 0)),
        scratch_shapes=[pltpu.VMEM((bn, C), jnp.float32)],
        compiler_params=pltpu.CompilerParams(
            dimension_semantics=("parallel", "arbitrary"),
            vmem_limit_bytes=int(32 << 20)),
    )(xt)
    return out.reshape(N, C, 1, 1)


def kernel(x):
    return _gem(x, p=3.0, eps=1e-6)


# speedup vs baseline: 1.5008x; 1.5008x over previous
import functools

import jax
import jax.numpy as jnp
from jax.experimental import pallas as pl
from jax.experimental.pallas import tpu as pltpu


def _gem_planes_kernel(x_ref, o_ref, *, hw, eps, inv_hw, inv_p):
    def body(i, acc):
        x = jnp.maximum(x_ref[i], jnp.float32(eps))
        return acc + x * x * x
    acc = jax.lax.fori_loop(
        0, hw, body, jnp.zeros(o_ref.shape, jnp.float32), unroll=True)
    o_ref[...] = jnp.power(acc * jnp.float32(inv_hw), jnp.float32(inv_p))


def _gem(x, p=3.0, eps=1e-6):
    N, C, H, W = x.shape
    HW = H * W
    xt = jnp.transpose(x, (2, 3, 0, 1)).reshape(HW, N, C)

    bc = 512
    while C % bc != 0:
        bc //= 2
    grid = C // bc

    kernel_fn = functools.partial(
        _gem_planes_kernel, hw=HW, eps=float(eps), inv_hw=1.0 / float(HW),
        inv_p=1.0 / float(p))
    out = pl.pallas_call(
        kernel_fn,
        out_shape=jax.ShapeDtypeStruct((N, C), x.dtype),
        grid=(grid,),
        in_specs=[pl.BlockSpec((HW, N, bc), lambda j: (0, 0, j))],
        out_specs=pl.BlockSpec((N, bc), lambda j: (0, j)),
        compiler_params=pltpu.CompilerParams(
            dimension_semantics=("parallel",),
            vmem_limit_bytes=int(32 << 20)),
    )(xt)
    return out.reshape(N, C, 1, 1)


def kernel(x):
    return _gem(x, p=3.0, eps=1e-6)
